# baseline (speedup 1.0000x reference)
import functools

import jax
import jax.numpy as jnp
from jax import lax
from jax.experimental import pallas as pl
from jax.experimental.pallas import tpu as pltpu
from jax.experimental.pallas import tpu_sc as plsc

N = 10000
E = 320000
G = 64

NC = 2
NS = 16
BB = 64
EPW = 10560
PADE = NC * NS * EPW
NSTAGE = 5
HALF = EPW // NSTAGE
BPH = HALF // BB
NPAD = 10240
ROWS_PER_TILE = NPAD // NS

RB = 1000
NRB = N // RB


def _dot(a, b):
    return jax.lax.dot_general(
        a, b, (((1,), (0,)), ((), ())),
        precision=jax.lax.Precision.HIGHEST,
        preferred_element_type=jnp.float32)


def _make_sc_aggregate(nchunks):
    mesh = plsc.VectorSubcoreMesh(core_axis_name="c", subcore_axis_name="s")
    scratch = [
        pltpu.VMEM_SHARED((NPAD, 128), jnp.float32),
        pltpu.VMEM((HALF,), jnp.int32),
        pltpu.VMEM((HALF,), jnp.int32),
        pltpu.VMEM((HALF,), jnp.float32),
        pltpu.VMEM((BB, 128), jnp.float32),
        pltpu.VMEM((BB, 128), jnp.float32),
        pltpu.VMEM((BB, 128), jnp.float32),
        pltpu.SemaphoreType.DMA,
        pltpu.SemaphoreType.DMA,
        pltpu.SemaphoreType.DMA,
        pltpu.SemaphoreType.DMA,
        pltpu.SemaphoreType.DMA,
        pltpu.SemaphoreType.DMA,
    ]

    @functools.partial(
        pl.kernel,
        out_type=jax.ShapeDtypeStruct((NC, nchunks, NPAD, 128), jnp.float32),
        mesh=mesh,
        scratch_types=scratch,
    )
    def k(*refs):
        xs = refs[:nchunks]
        (src_hbm, dst_hbm, w_hbm, out_hbm, acc, sidx, didx, wv,
         r0, r1, r2, s0, s1, s2, q0, q1, q2) = refs[nchunks:]
        cc = lax.axis_index("c")
        t = lax.axis_index("s")
        wid = cc * NS + t
        rows = (r0, r1, r2)
        sems = (s0, s1, s2)
        qsems = (q0, q1, q2)

        for c in range(nchunks):
            @pl.loop(0, BB)
            def _(r):
                for j in range(8):
                    r0[r, pl.ds(j * 16, 16)] = jnp.zeros((16,), jnp.float32)

            @pl.loop(0, ROWS_PER_TILE // BB)
            def _(z):
                pltpu.sync_copy(r0, acc.at[pl.ds(t * ROWS_PER_TILE + z * BB,
                                                 BB)])
            plsc.subcore_barrier()

            @pl.loop(0, NSTAGE)
            def _(h):
                off = wid * EPW + h * HALF
                pltpu.sync_copy(src_hbm.at[pl.ds(off, HALF)], sidx)
                pltpu.sync_copy(dst_hbm.at[pl.ds(off, HALF)], didx)
                pltpu.sync_copy(w_hbm.at[pl.ds(off, HALF)], wv)
                for b in range(2):
                    pltpu.async_copy(xs[c].at[sidx.at[pl.ds(b * BB, BB)]],
                                     rows[b], sems[b])

                @pl.loop(0, BPH // 3)
                def _(z3):
                    for b in range(3):
                        kk = z3 * 3 + b
                        base = kk * BB
                        nb = (b + 2) % 3
                        pltpu.make_async_copy(
                            xs[c].at[sidx.at[pl.ds(base, BB)]],
                            rows[b], sems[b]).wait()

                        @pl.loop(0, BB // 16)
                        def _(g):
                            wvec = wv[pl.ds(base + g * 16, 16)]
                            for i2 in range(16):
                                ws = wvec.at[
                                    jnp.full((16,), i2, jnp.int32)].get(
                                        mode="promise_in_bounds")
                                for j in range(8):
                                    sl = (g * 16 + i2, pl.ds(j * 16, 16))
                                    rows[b][sl] = rows[b][sl] * ws

                        pltpu.async_copy(rows[b],
                                         acc.at[didx.at[pl.ds(base, BB)]],
                                         qsems[b], add=True)

                        @pl.when(kk + 2 < BPH)
                        def _():
                            @pl.when(kk >= 1)
                            def _():
                                pltpu.make_async_copy(
                                    rows[nb],
                                    acc.at[didx.at[pl.ds((kk - 1) * BB, BB)]],
                                    qsems[nb]).wait()
                            pltpu.async_copy(
                                xs[c].at[sidx.at[pl.ds(base + 2 * BB, BB)]],
                                rows[nb], sems[nb])

                for b in range(3):
                    pltpu.make_async_copy(
                        rows[b],
                        acc.at[didx.at[pl.ds((BPH - 3 + b) * BB, BB)]],
                        qsems[b]).wait()

            plsc.subcore_barrier()
            pltpu.sync_copy(acc.at[pl.ds(t * ROWS_PER_TILE, ROWS_PER_TILE)],
                            out_hbm.at[cc, c,
                                       pl.ds(t * ROWS_PER_TILE,
                                             ROWS_PER_TILE)])
            plsc.subcore_barrier()

    return k


def _layer1_body(a0, a1, x, wrel, wroot, b1, h1s):
    out = _dot(a0[...] + a1[...], wrel[...]) + _dot(x[...], wroot[...])
    h1s[0] = jnp.maximum(out + b1[0], 0.0)


def _layer2_body(a0, a1, h1s, wrel, wroot, b2, t2s, sums, sumsq):
    k = pl.program_id(2)
    contrib = _dot(a0[0] + a1[0], wrel[...]) + _dot(h1s[0], wroot[...])

    @pl.when(k == 0)
    def _():
        t2s[0] = contrib + b2[0]

    @pl.when(k > 0)
    def _():
        t2s[0] = t2s[0] + contrib

    @pl.when(k == 3)
    def _():
        r = pl.program_id(1)
        tb = t2s[0].reshape(RB // 8, 8, 128)
        part = jnp.sum(tb, axis=0)
        partsq = jnp.sum(tb * tb, axis=0)

        @pl.when(r == 0)
        def _():
            sums[0] = part
            sumsq[0] = partsq

        @pl.when(r > 0)
        def _():
            sums[0] = sums[0] + part
            sumsq[0] = sumsq[0] + partsq


def _layer3pre_body(t2s, sums, sumsq, gamma, beta, wrel, wroot, b3, y3s, r3s):
    k = pl.program_id(2)
    total = jnp.sum(sums[0], axis=0) / N
    totsq = jnp.sum(sumsq[0], axis=0) / N
    var = totsq - total * total
    scale = gamma[0, 0] * jax.lax.rsqrt(var + 1e-5)
    shift = beta[0, 0] - total * scale
    h2 = jnp.maximum(t2s[0] * scale + shift, 0.0)
    y = _dot(h2, wrel[...])
    rt = _dot(h2, wroot[...])

    @pl.when(k == 0)
    def _():
        y3s[0] = y
        r3s[0] = rt + b3[0]

    @pl.when(k > 0)
    def _():
        y3s[0] = y3s[0] + y
        r3s[0] = r3s[0] + rt


def _pool_body(a0, a1, r3s, batch, psums, cnts):
    co = pl.program_id(0)
    r = pl.program_id(1)
    h3 = jnp.maximum(a0[0] + a1[0] + r3s[0], 0.0)
    bidx = batch[0, 0]
    pt = (lax.broadcasted_iota(jnp.int32, (G, RB), 0)
          == bidx[None, :]).astype(jnp.float32)
    part = _dot(pt, h3)

    @pl.when(r == 0)
    def _():
        psums[0] = part

    @pl.when(r > 0)
    def _():
        psums[0] = psums[0] + part

    @pl.when(co == 0)
    def _():
        cmat = _dot(pt, jnp.ones((RB, 128), jnp.float32))

        @pl.when(r == 0)
        def _():
            cnts[...] = cmat

        @pl.when(r > 0)
        def _():
            cnts[...] = cnts[...] + cmat


def _head_body(psums, cnts, wl1t, bl1, wl2, bl2, out):
    cm = jnp.maximum(cnts[...], 1.0)
    pooled = jnp.concatenate([psums[0] / cm, psums[1] / cm], axis=1)
    h = jnp.maximum(_dot(pooled, wl1t[...]) + bl1[...], 0.0)
    s = jnp.sum(h * wl2[...], axis=1, keepdims=True) + bl2[0]
    out[...] = jnp.maximum(s, 0.0)


def _rb_spec():
    return pl.BlockSpec((RB, 128), lambda c, r: (r, 0))


def kernel(x, edge_index, edge_attr, batch,
           W1_rel, b1, W1_root, W2_rel, b2, W2_root, W3_rel, b3, W3_root,
           gamma, beta, Wl1, bl1, Wl2, bl2):
    f32 = jnp.float32
    pad = PADE - E
    src = jnp.pad(edge_index[0], (0, pad))
    dst = jnp.pad(edge_index[1], (0, pad))
    ew = jnp.pad(edge_attr, (0, pad))

    agg1 = _make_sc_aggregate(1)(x, src, dst, ew)[:, :, :N]

    h1s = pl.pallas_call(
        _layer1_body,
        grid=(4, NRB),
        in_specs=[
            _rb_spec(), _rb_spec(), _rb_spec(),
            pl.BlockSpec((128, 128), lambda c, r: (0, c)),
            pl.BlockSpec((128, 128), lambda c, r: (0, c)),
            pl.BlockSpec((1, 1, 128), lambda c, r: (c, 0, 0)),
        ],
        out_specs=pl.BlockSpec((1, RB, 128), lambda c, r: (c, r, 0)),
        out_shape=jax.ShapeDtypeStruct((4, N, 128), f32),
    )(agg1[0, 0], agg1[1, 0], x, W1_rel.T, W1_root.T,
      b1.reshape(4, 1, 128))

    agg2 = _make_sc_aggregate(4)(h1s[0], h1s[1], h1s[2], h1s[3],
                                 src, dst, ew)[:, :, :N]

    c3 = pl.BlockSpec((1, RB, 128), lambda c, r, k: (k, r, 0))
    t2s, sums, sumsq = pl.pallas_call(
        _layer2_body,
        grid=(4, NRB, 4),
        in_specs=[
            c3, c3, c3,
            pl.BlockSpec((128, 128), lambda c, r, k: (k, c)),
            pl.BlockSpec((128, 128), lambda c, r, k: (k, c)),
            pl.BlockSpec((1, 1, 128), lambda c, r, k: (c, 0, 0)),
        ],
        out_specs=[
            pl.BlockSpec((1, RB, 128), lambda c, r, k: (c, r, 0)),
            pl.BlockSpec((1, 8, 128), lambda c, r, k: (c, 0, 0)),
            pl.BlockSpec((1, 8, 128), lambda c, r, k: (c, 0, 0)),
        ],
        out_shape=[
            jax.ShapeDtypeStruct((4, N, 128), f32),
            jax.ShapeDtypeStruct((4, 8, 128), f32),
            jax.ShapeDtypeStruct((4, 8, 128), f32),
        ],
    )(agg2[0], agg2[1], h1s, W2_rel.T, W2_root.T, b2.reshape(4, 1, 128))

    k3 = pl.BlockSpec((1, RB, 128), lambda co, r, k: (k, r, 0))
    st3 = pl.BlockSpec((1, 8, 128), lambda co, r, k: (k, 0, 0))
    g3 = pl.BlockSpec((1, 1, 128), lambda co, r, k: (k, 0, 0))
    y3s, r3s = pl.pallas_call(
        _layer3pre_body,
        grid=(2, NRB, 4),
        in_specs=[
            k3, st3, st3, g3, g3,
            pl.BlockSpec((128, 128), lambda co, r, k: (k, co)),
            pl.BlockSpec((128, 128), lambda co, r, k: (k, co)),
            pl.BlockSpec((1, 1, 128), lambda co, r, k: (co, 0, 0)),
        ],
        out_specs=[
            pl.BlockSpec((1, RB, 128), lambda co, r, k: (co, r, 0)),
            pl.BlockSpec((1, RB, 128), lambda co, r, k: (co, r, 0)),
        ],
        out_shape=[
            jax.ShapeDtypeStruct((2, N, 128), f32),
            jax.ShapeDtypeStruct((2, N, 128), f32),
        ],
    )(t2s, sums, sumsq, gamma.reshape(4, 1, 128), beta.reshape(4, 1, 128),
      W3_rel.T, W3_root.T, b3.reshape(2, 1, 128))

    agg3 = _make_sc_aggregate(2)(y3s[0], y3s[1], src, dst, ew)[:, :, :N]

    p2 = pl.BlockSpec((1, RB, 128), lambda co, r: (co, r, 0))
    psums, cnts = pl.pallas_call(
        _pool_body,
        grid=(2, NRB),
        in_specs=[
            p2, p2, p2,
            pl.BlockSpec((1, 1, RB), lambda co, r: (r, 0, 0)),
        ],
        out_specs=[
            pl.BlockSpec((1, G, 128), lambda co, r: (co, 0, 0)),
            pl.BlockSpec((G, 128), lambda co, r: (0, 0)),
        ],
        out_shape=[
            jax.ShapeDtypeStruct((2, G, 128), f32),
            jax.ShapeDtypeStruct((G, 128), f32),
        ],
    )(agg3[0], agg3[1], r3s, batch.reshape(NRB, 1, RB))

    out = pl.pallas_call(
        _head_body,
        out_shape=jax.ShapeDtypeStruct((G, 1), f32),
    )(psums, cnts, Wl1.T, bl1, Wl2, bl2)
    return out


# speedup vs baseline: 1.8860x; 1.8860x over previous
import functools

import jax
import jax.numpy as jnp
from jax import lax
from jax.experimental import pallas as pl
from jax.experimental.pallas import tpu as pltpu
from jax.experimental.pallas import tpu_sc as plsc

N = 10000
E = 320000
G = 64

NC = 2
NS = 16
BB = 64
EPW = 10240
PADE = NC * NS * EPW
HALF = EPW // 2
BPH = HALF // BB
NPAD = 10240
ROWS_PER_TILE = NPAD // NS

RB = 1000
NRB = N // RB


def _dot(a, b):
    return jax.lax.dot_general(
        a, b, (((1,), (0,)), ((), ())),
        precision=jax.lax.Precision.HIGHEST,
        preferred_element_type=jnp.float32)


def _make_sc_aggregate(nchunks):
    mesh = plsc.VectorSubcoreMesh(core_axis_name="c", subcore_axis_name="s")
    scratch = [
        pltpu.VMEM_SHARED((NPAD, 128), jnp.float32),
        pltpu.VMEM((HALF,), jnp.int32),
        pltpu.VMEM((HALF,), jnp.int32),
        pltpu.VMEM((HALF,), jnp.float32),
        pltpu.VMEM((BB, 128), jnp.float32),
        pltpu.VMEM((BB, 128), jnp.float32),
        pltpu.SemaphoreType.DMA,
        pltpu.SemaphoreType.DMA,
    ]

    @functools.partial(
        pl.kernel,
        out_type=jax.ShapeDtypeStruct((NC, nchunks, NPAD, 128), jnp.float32),
        mesh=mesh,
        scratch_types=scratch,
    )
    def k(*refs):
        xs = refs[:nchunks]
        (src_hbm, dst_hbm, w_hbm, out_hbm, acc, sidx, didx, wv,
         r0, r1, s0, s1) = refs[nchunks:]
        cc = lax.axis_index("c")
        t = lax.axis_index("s")
        wid = cc * NS + t
        rows = (r0, r1)
        sems = (s0, s1)

        for c in range(nchunks):
            @pl.loop(0, BB)
            def _(r):
                for j in range(8):
                    r0[r, pl.ds(j * 16, 16)] = jnp.zeros((16,), jnp.float32)

            @pl.loop(0, ROWS_PER_TILE // BB)
            def _(z):
                pltpu.sync_copy(r0, acc.at[pl.ds(t * ROWS_PER_TILE + z * BB,
                                                 BB)])
            plsc.subcore_barrier()

            @pl.loop(0, 2)
            def _(h):
                off = wid * EPW + h * HALF
                pltpu.sync_copy(src_hbm.at[pl.ds(off, HALF)], sidx)
                pltpu.sync_copy(dst_hbm.at[pl.ds(off, HALF)], didx)
                pltpu.sync_copy(w_hbm.at[pl.ds(off, HALF)], wv)
                for b in range(2):
                    pltpu.async_copy(xs[c].at[sidx.at[pl.ds(b * BB, BB)]],
                                     rows[b], sems[b])

                @pl.loop(0, BPH // 2)
                def _(z2):
                    for b in range(2):
                        kk = z2 * 2 + b
                        base = kk * BB
                        pltpu.make_async_copy(
                            xs[c].at[sidx.at[pl.ds(base, BB)]],
                            rows[b], sems[b]).wait()

                        @pl.loop(0, BB // 16)
                        def _(g):
                            wvec = wv[pl.ds(base + g * 16, 16)]
                            for i2 in range(16):
                                ws = wvec.at[
                                    jnp.full((16,), i2, jnp.int32)].get(
                                        mode="promise_in_bounds")
                                for j in range(8):
                                    sl = (g * 16 + i2, pl.ds(j * 16, 16))
                                    rows[b][sl] = rows[b][sl] * ws

                        pltpu.sync_copy(rows[b],
                                        acc.at[didx.at[pl.ds(base, BB)]],
                                        add=True)

                        @pl.when(kk + 2 < BPH)
                        def _():
                            pltpu.async_copy(
                                xs[c].at[sidx.at[pl.ds(base + 2 * BB, BB)]],
                                rows[b], sems[b])

            plsc.subcore_barrier()
            pltpu.sync_copy(acc.at[pl.ds(t * ROWS_PER_TILE, ROWS_PER_TILE)],
                            out_hbm.at[cc, c,
                                       pl.ds(t * ROWS_PER_TILE,
                                             ROWS_PER_TILE)])
            plsc.subcore_barrier()

    return k


def _layer1_body(a0, a1, x, wrel, wroot, b1, h1s):
    out = _dot(a0[...] + a1[...], wrel[...]) + _dot(x[...], wroot[...])
    h1s[0] = jnp.maximum(out + b1[0], 0.0)


def _layer2_body(a0, a1, h1s, wrel, wroot, b2, t2s, sums, sumsq):
    k = pl.program_id(2)
    contrib = _dot(a0[0] + a1[0], wrel[...]) + _dot(h1s[0], wroot[...])

    @pl.when(k == 0)
    def _():
        t2s[0] = contrib + b2[0]

    @pl.when(k > 0)
    def _():
        t2s[0] = t2s[0] + contrib

    @pl.when(k == 3)
    def _():
        r = pl.program_id(1)
        tb = t2s[0].reshape(RB // 8, 8, 128)
        part = jnp.sum(tb, axis=0)
        partsq = jnp.sum(tb * tb, axis=0)

        @pl.when(r == 0)
        def _():
            sums[0] = part
            sumsq[0] = partsq

        @pl.when(r > 0)
        def _():
            sums[0] = sums[0] + part
            sumsq[0] = sumsq[0] + partsq


def _layer3pre_body(t2s, sums, sumsq, gamma, beta, wrel, wroot, b3, y3s, r3s):
    k = pl.program_id(2)
    total = jnp.sum(sums[0], axis=0) / N
    totsq = jnp.sum(sumsq[0], axis=0) / N
    var = totsq - total * total
    scale = gamma[0, 0] * jax.lax.rsqrt(var + 1e-5)
    shift = beta[0, 0] - total * scale
    h2 = jnp.maximum(t2s[0] * scale + shift, 0.0)
    y = _dot(h2, wrel[...])
    rt = _dot(h2, wroot[...])

    @pl.when(k == 0)
    def _():
        y3s[0] = y
        r3s[0] = rt + b3[0]

    @pl.when(k > 0)
    def _():
        y3s[0] = y3s[0] + y
        r3s[0] = r3s[0] + rt


def _pool_body(a0, a1, r3s, batch, psums, cnts):
    co = pl.program_id(0)
    r = pl.program_id(1)
    h3 = jnp.maximum(a0[0] + a1[0] + r3s[0], 0.0)
    bidx = batch[0, 0]
    pt = (lax.broadcasted_iota(jnp.int32, (G, RB), 0)
          == bidx[None, :]).astype(jnp.float32)
    part = _dot(pt, h3)

    @pl.when(r == 0)
    def _():
        psums[0] = part

    @pl.when(r > 0)
    def _():
        psums[0] = psums[0] + part

    @pl.when(co == 0)
    def _():
        cmat = _dot(pt, jnp.ones((RB, 128), jnp.float32))

        @pl.when(r == 0)
        def _():
            cnts[...] = cmat

        @pl.when(r > 0)
        def _():
            cnts[...] = cnts[...] + cmat


def _head_body(psums, cnts, wl1t, bl1, wl2, bl2, out):
    cm = jnp.maximum(cnts[...], 1.0)
    pooled = jnp.concatenate([psums[0] / cm, psums[1] / cm], axis=1)
    h = jnp.maximum(_dot(pooled, wl1t[...]) + bl1[...], 0.0)
    s = jnp.sum(h * wl2[...], axis=1, keepdims=True) + bl2[0]
    out[...] = jnp.maximum(s, 0.0)


def _rb_spec():
    return pl.BlockSpec((RB, 128), lambda c, r: (r, 0))


def kernel(x, edge_index, edge_attr, batch,
           W1_rel, b1, W1_root, W2_rel, b2, W2_root, W3_rel, b3, W3_root,
           gamma, beta, Wl1, bl1, Wl2, bl2):
    f32 = jnp.float32
    pad = PADE - E
    src = jnp.pad(edge_index[0], (0, pad))
    dst = jnp.pad(edge_index[1], (0, pad))
    ew = jnp.pad(edge_attr, (0, pad))

    agg1 = _make_sc_aggregate(1)(x, src, dst, ew)[:, :, :N]

    h1s = pl.pallas_call(
        _layer1_body,
        grid=(4, NRB),
        in_specs=[
            _rb_spec(), _rb_spec(), _rb_spec(),
            pl.BlockSpec((128, 128), lambda c, r: (0, c)),
            pl.BlockSpec((128, 128), lambda c, r: (0, c)),
            pl.BlockSpec((1, 1, 128), lambda c, r: (c, 0, 0)),
        ],
        out_specs=pl.BlockSpec((1, RB, 128), lambda c, r: (c, r, 0)),
        out_shape=jax.ShapeDtypeStruct((4, N, 128), f32),
    )(agg1[0, 0], agg1[1, 0], x, W1_rel.T, W1_root.T,
      b1.reshape(4, 1, 128))

    agg2 = _make_sc_aggregate(4)(h1s[0], h1s[1], h1s[2], h1s[3],
                                 src, dst, ew)[:, :, :N]

    c3 = pl.BlockSpec((1, RB, 128), lambda c, r, k: (k, r, 0))
    t2s, sums, sumsq = pl.pallas_call(
        _layer2_body,
        grid=(4, NRB, 4),
        in_specs=[
            c3, c3, c3,
            pl.BlockSpec((128, 128), lambda c, r, k: (k, c)),
            pl.BlockSpec((128, 128), lambda c, r, k: (k, c)),
            pl.BlockSpec((1, 1, 128), lambda c, r, k: (c, 0, 0)),
        ],
        out_specs=[
            pl.BlockSpec((1, RB, 128), lambda c, r, k: (c, r, 0)),
            pl.BlockSpec((1, 8, 128), lambda c, r, k: (c, 0, 0)),
            pl.BlockSpec((1, 8, 128), lambda c, r, k: (c, 0, 0)),
        ],
        out_shape=[
            jax.ShapeDtypeStruct((4, N, 128), f32),
            jax.ShapeDtypeStruct((4, 8, 128), f32),
            jax.ShapeDtypeStruct((4, 8, 128), f32),
        ],
    )(agg2[0], agg2[1], h1s, W2_rel.T, W2_root.T, b2.reshape(4, 1, 128))

    k3 = pl.BlockSpec((1, RB, 128), lambda co, r, k: (k, r, 0))
    st3 = pl.BlockSpec((1, 8, 128), lambda co, r, k: (k, 0, 0))
    g3 = pl.BlockSpec((1, 1, 128), lambda co, r, k: (k, 0, 0))
    y3s, r3s = pl.pallas_call(
        _layer3pre_body,
        grid=(2, NRB, 4),
        in_specs=[
            k3, st3, st3, g3, g3,
            pl.BlockSpec((128, 128), lambda co, r, k: (k, co)),
            pl.BlockSpec((128, 128), lambda co, r, k: (k, co)),
            pl.BlockSpec((1, 1, 128), lambda co, r, k: (co, 0, 0)),
        ],
        out_specs=[
            pl.BlockSpec((1, RB, 128), lambda co, r, k: (co, r, 0)),
            pl.BlockSpec((1, RB, 128), lambda co, r, k: (co, r, 0)),
        ],
        out_shape=[
            jax.ShapeDtypeStruct((2, N, 128), f32),
            jax.ShapeDtypeStruct((2, N, 128), f32),
        ],
    )(t2s, sums, sumsq, gamma.reshape(4, 1, 128), beta.reshape(4, 1, 128),
      W3_rel.T, W3_root.T, b3.reshape(2, 1, 128))

    agg3 = _make_sc_aggregate(2)(y3s[0], y3s[1], src, dst, ew)[:, :, :N]

    p2 = pl.BlockSpec((1, RB, 128), lambda co, r: (co, r, 0))
    psums, cnts = pl.pallas_call(
        _pool_body,
        grid=(2, NRB),
        in_specs=[
            p2, p2, p2,
            pl.BlockSpec((1, 1, RB), lambda co, r: (r, 0, 0)),
        ],
        out_specs=[
            pl.BlockSpec((1, G, 128), lambda co, r: (co, 0, 0)),
            pl.BlockSpec((G, 128), lambda co, r: (0, 0)),
        ],
        out_shape=[
            jax.ShapeDtypeStruct((2, G, 128), f32),
            jax.ShapeDtypeStruct((G, 128), f32),
        ],
    )(agg3[0], agg3[1], r3s, batch.reshape(NRB, 1, RB))

    out = pl.pallas_call(
        _head_body,
        out_shape=jax.ShapeDtypeStruct((G, 1), f32),
    )(psums, cnts, Wl1.T, bl1, Wl2, bl2)
    return out
